# baseline (speedup 1.0000x reference)
import jax
import jax.numpy as jnp
from jax.experimental import pallas as pl

_BLOCK = 2048


def _onehot(d):
    k = jax.lax.broadcasted_iota(jnp.int32, (d, 4), 0)
    j = jax.lax.broadcasted_iota(jnp.int32, (d, 4), 1)
    hit = lambda kk, jj: ((k == kk) & (j == jj)).astype(jnp.float32)
    return hit(52, 0) - hit(55, 1) - hit(9, 2) - hit(12, 3)


def _angle_prior_kernel(x_ref, out_ref):
    g = jnp.dot(x_ref[...], _onehot(x_ref.shape[1]),
                preferred_element_type=jnp.float32,
                precision=jax.lax.Precision.HIGHEST)
    e = jnp.exp(g)
    out_ref[...] = e * e


def kernel(pose):
    n, d = pose.shape
    return pl.pallas_call(
        _angle_prior_kernel,
        grid=(n // _BLOCK,),
        in_specs=[pl.BlockSpec((_BLOCK, d), lambda i: (i, 0))],
        out_specs=pl.BlockSpec((_BLOCK, 4), lambda i: (i, 0)),
        out_shape=jax.ShapeDtypeStruct((n, 4), pose.dtype),
    )(pose)
